# Initial kernel scaffold; baseline (speedup 1.0000x reference)
import jax
import jax.numpy as jnp
from jax.experimental import pallas as pl


def kernel(node_embeds, edge_index_rel0, edge_index_rel1, edge_index_rel2, rel_emb_rel0, rel_emb_rel1, rel_emb_rel2, rel_emb_no_relation):
    raise NotImplementedError("write your pallas kernel here")


# speedup vs baseline: 1.5844x; 1.5844x over previous
import functools

import jax
import jax.numpy as jnp
from jax import lax
from jax.experimental import pallas as pl
from jax.experimental.pallas import tpu as pltpu
from jax.experimental.pallas import tpu_sc as plsc

N_NODES = 10000
D = 128
E = 100000
NC = 2
NS = 16
NW = NC * NS
PER_W = 3200
E_PAD = NW * PER_W
C = 80
NCHUNK = PER_W // C
NJ = D // 16


def _body(node_h, s0, d0, s1, d1, s2, d2, rel_h,
          o0, o1, o2,
          is0, id0, is1, id1, is2, id2, rel_v, out_v,
          sbufA, dbufA, sbufB, dbufB, semA, semB):
    cid = lax.axis_index("c")
    sid = lax.axis_index("s")
    wid = sid * NC + cid
    base = wid * PER_W

    pltpu.sync_copy(rel_h, rel_v)
    idx_refs = ((s0, is0), (d0, id0), (s1, is1), (d1, id1), (s2, is2), (d2, id2))
    for hbm, vmem in idx_refs:
        pltpu.sync_copy(hbm.at[pl.ds(base, PER_W)], vmem)

    lane = lax.iota(jnp.int32, 16)
    perms = [lax.rem(lane + s, 16) for s in (8, 4, 2, 1)]

    gdn = lax.GatherDimensionNumbers(
        offset_dims=(), collapsed_slice_dims=(0,), start_index_map=(0,))

    def lanesum(v):
        for p in perms:
            v = v + lax.gather(v, p[:, None], gdn, slice_sizes=(1,),
                               mode=lax.GatherScatterMode.PROMISE_IN_BOUNDS)
        return v

    for r, (isv, idv, oh) in enumerate(((is0, id0, o0), (is1, id1, o1), (is2, id2, o2))):
        relregs = [rel_v[r, pl.ds(16 * j, 16)] for j in range(NJ)]

        pltpu.async_copy(node_h.at[isv.at[pl.ds(0, C)]], sbufA, semA)
        pltpu.async_copy(node_h.at[idv.at[pl.ds(0, C)]], dbufA, semA)

        @pl.loop(0, NCHUNK, step=2)
        def _(c0):
            bufs = ((sbufA, dbufA, semA, sbufB, dbufB, semB),
                    (sbufB, dbufB, semB, sbufA, dbufA, semA))
            for b, (sb, db, sem, osb, odb, osem) in enumerate(bufs):
                c = c0 + b
                pltpu.make_async_copy(node_h.at[isv.at[pl.ds(0, C)]], sb, sem).wait()
                pltpu.make_async_copy(node_h.at[idv.at[pl.ds(0, C)]], db, sem).wait()

                @pl.when(c + 1 < NCHUNK)
                def _():
                    off = (c + 1) * C
                    pltpu.async_copy(node_h.at[isv.at[pl.ds(off, C)]], osb, osem)
                    pltpu.async_copy(node_h.at[idv.at[pl.ds(off, C)]], odb, osem)

                def grp(g, carry):
                    acc = jnp.zeros((16,), jnp.float32)
                    for e in range(16):
                        row = g * 16 + e
                        p = sb[row, pl.ds(0, 16)] * db[row, pl.ds(0, 16)] * relregs[0]
                        for j in range(1, NJ):
                            p = p + (sb[row, pl.ds(16 * j, 16)]
                                     * db[row, pl.ds(16 * j, 16)] * relregs[j])
                        acc = jnp.where(lane == e, lanesum(p), acc)
                    out_v[pl.ds(c * C + g * 16, 16)] = acc
                    return carry

                lax.fori_loop(0, C // 16, grp, 0)

        pltpu.sync_copy(out_v, oh.at[pl.ds(base, PER_W)])


@jax.jit
def _dis_mult_sc(node_embeds, s0, d0, s1, d1, s2, d2, rel_all):
    f = pl.kernel(
        _body,
        out_type=[jax.ShapeDtypeStruct((E_PAD,), jnp.float32)] * 3,
        mesh=plsc.VectorSubcoreMesh(core_axis_name="c", subcore_axis_name="s",
                                    num_cores=NC, num_subcores=NS),
        scratch_types=[pltpu.VMEM((PER_W,), jnp.int32) for _ in range(6)]
        + [pltpu.VMEM((3, D), jnp.float32), pltpu.VMEM((PER_W,), jnp.float32)]
        + [pltpu.VMEM((C, D), jnp.float32) for _ in range(4)]
        + [pltpu.SemaphoreType.DMA, pltpu.SemaphoreType.DMA],
    )
    return f(node_embeds, s0, d0, s1, d1, s2, d2, rel_all)


def kernel(node_embeds, edge_index_rel0, edge_index_rel1, edge_index_rel2,
           rel_emb_rel0, rel_emb_rel1, rel_emb_rel2, rel_emb_no_relation):
    pad = E_PAD - E
    flat = []
    for ei in (edge_index_rel0, edge_index_rel1, edge_index_rel2):
        flat.append(jnp.pad(ei[0], (0, pad)))
        flat.append(jnp.pad(ei[1], (0, pad)))
    rel_all = jnp.concatenate([rel_emb_rel0, rel_emb_rel1, rel_emb_rel2], axis=0)
    outs = _dis_mult_sc(node_embeds, *flat, rel_all)
    return tuple(o[:E] for o in outs)


# speedup vs baseline: 2.7816x; 1.7556x over previous
import functools

import numpy as np
import jax
import jax.numpy as jnp
from jax import lax
from jax.experimental import pallas as pl
from jax.experimental.pallas import tpu as pltpu
from jax.experimental.pallas import tpu_sc as plsc

N_NODES = 10000
D = 128
DW = D // 2
E = 100000
NC = 2
NS = 16
NW = NC * NS
PER_W = 3200
E_PAD = NW * PER_W
C = 160
NCHUNK = PER_W // C
NT = DW // 16

_HI_MASK = np.int32(-65536)


def _body(node_h, s0, d0, s1, d1, s2, d2, rel_h,
          o0, o1, o2,
          is0, id0, is1, id1, is2, id2, rel_v, out_v,
          sbufA, dbufA, sbufB, dbufB, semA, semB):
    cid = lax.axis_index("c")
    sid = lax.axis_index("s")
    wid = sid * NC + cid
    base = wid * PER_W

    pltpu.sync_copy(rel_h, rel_v)
    idx_refs = ((s0, is0), (d0, id0), (s1, is1), (d1, id1), (s2, is2), (d2, id2))
    for hbm, vmem in idx_refs:
        pltpu.sync_copy(hbm.at[pl.ds(base, PER_W)], vmem)

    lane = lax.iota(jnp.int32, 16)
    perms = [lax.rem(lane + s, 16) for s in (8, 4, 2, 1)]
    gdn = lax.GatherDimensionNumbers(
        offset_dims=(), collapsed_slice_dims=(0,), start_index_map=(0,))

    def lanesum(v):
        for p in perms:
            v = v + lax.gather(v, p[:, None], gdn, slice_sizes=(1,),
                               mode=lax.GatherScatterMode.PROMISE_IN_BOUNDS)
        return v

    def unpack(w):
        lo = plsc.bitcast(lax.shift_left(w, 16), jnp.float32)
        hi = plsc.bitcast(lax.bitwise_and(w, _HI_MASK), jnp.float32)
        return lo, hi

    for r, (isv, idv, oh) in enumerate(((is0, id0, o0), (is1, id1, o1), (is2, id2, o2))):
        relregs = [rel_v[r, pl.ds(16 * j, 16)] for j in range(2 * NT)]

        pltpu.async_copy(node_h.at[isv.at[pl.ds(0, C)]], sbufA, semA)
        pltpu.async_copy(node_h.at[idv.at[pl.ds(0, C)]], dbufA, semA)

        @pl.loop(0, NCHUNK, step=2)
        def _(c0):
            bufs = ((sbufA, dbufA, semA, sbufB, dbufB, semB),
                    (sbufB, dbufB, semB, sbufA, dbufA, semA))
            for b, (sb, db, sem, osb, odb, osem) in enumerate(bufs):
                c = c0 + b
                pltpu.make_async_copy(node_h.at[isv.at[pl.ds(0, C)]], sb, sem).wait()
                pltpu.make_async_copy(node_h.at[idv.at[pl.ds(0, C)]], db, sem).wait()

                @pl.when(c + 1 < NCHUNK)
                def _():
                    off = (c + 1) * C
                    pltpu.async_copy(node_h.at[isv.at[pl.ds(off, C)]], osb, osem)
                    pltpu.async_copy(node_h.at[idv.at[pl.ds(off, C)]], odb, osem)

                def grp(g, carry):
                    acc = jnp.zeros((16,), jnp.float32)
                    for e in range(16):
                        row = g * 16 + e
                        p = None
                        for t in range(NT):
                            slo, shi = unpack(sb[row, pl.ds(16 * t, 16)])
                            dlo, dhi = unpack(db[row, pl.ds(16 * t, 16)])
                            q = slo * dlo * relregs[2 * t] + shi * dhi * relregs[2 * t + 1]
                            p = q if p is None else p + q
                        acc = jnp.where(lane == e, lanesum(p), acc)
                    out_v[pl.ds(c * C + g * 16, 16)] = acc
                    return carry

                lax.fori_loop(0, C // 16, grp, 0)

        pltpu.sync_copy(out_v, oh.at[pl.ds(base, PER_W)])


@jax.jit
def _dis_mult_sc(node_packed, s0, d0, s1, d1, s2, d2, rel_prep):
    f = pl.kernel(
        _body,
        out_type=[jax.ShapeDtypeStruct((E_PAD,), jnp.float32)] * 3,
        mesh=plsc.VectorSubcoreMesh(core_axis_name="c", subcore_axis_name="s",
                                    num_cores=NC, num_subcores=NS),
        scratch_types=[pltpu.VMEM((PER_W,), jnp.int32) for _ in range(6)]
        + [pltpu.VMEM((3, D), jnp.float32), pltpu.VMEM((PER_W,), jnp.float32)]
        + [pltpu.VMEM((C, DW), jnp.int32) for _ in range(4)]
        + [pltpu.SemaphoreType.DMA, pltpu.SemaphoreType.DMA],
        compiler_params=pltpu.CompilerParams(needs_layout_passes=False,
                                             use_tc_tiling_on_sc=False),
    )
    return f(node_packed, s0, d0, s1, d1, s2, d2, rel_prep)


def kernel(node_embeds, edge_index_rel0, edge_index_rel1, edge_index_rel2,
           rel_emb_rel0, rel_emb_rel1, rel_emb_rel2, rel_emb_no_relation):
    pad = E_PAD - E
    flat = []
    for ei in (edge_index_rel0, edge_index_rel1, edge_index_rel2):
        flat.append(jnp.pad(ei[0], (0, pad)))
        flat.append(jnp.pad(ei[1], (0, pad)))
    node_bf = node_embeds.astype(jnp.bfloat16)
    node_packed = lax.bitcast_convert_type(
        node_bf.reshape(N_NODES, DW, 2), jnp.int32)
    rel_all = jnp.concatenate([rel_emb_rel0, rel_emb_rel1, rel_emb_rel2], axis=0)
    rel_prep = rel_all.reshape(3, NT, 16, 2).transpose(0, 1, 3, 2).reshape(3, D)
    outs = _dis_mult_sc(node_packed, *flat, rel_prep)
    return tuple(o[:E] for o in outs)


# speedup vs baseline: 7.0123x; 2.5210x over previous
import functools

import numpy as np
import jax
import jax.numpy as jnp
from jax import lax
from jax.experimental import pallas as pl
from jax.experimental.pallas import tpu as pltpu
from jax.experimental.pallas import tpu_sc as plsc

N_NODES = 10000
D = 128
DW = D // 2
E = 100000
NC = 2
NS = 16
NW = NC * NS
PER_W = 3200
E_PAD = NW * PER_W
C = 160
NCHUNK = PER_W // C
NT = DW // 16

_HI_MASK = np.int32(-65536)


def _body(node_h, s0, d0, s1, d1, s2, d2, rel_h,
          o0, o1, o2,
          is0, id0, is1, id1, is2, id2, rel_v, out_v,
          sbufA, dbufA, sbufB, dbufB, shared_tab, semA, semB):
    cid = lax.axis_index("c")
    sid = lax.axis_index("s")
    wid = sid * NC + cid
    base = wid * PER_W

    @pl.when(sid == 0)
    def _():
        pltpu.sync_copy(node_h, shared_tab)

    pltpu.sync_copy(rel_h, rel_v)
    idx_refs = ((s0, is0), (d0, id0), (s1, is1), (d1, id1), (s2, is2), (d2, id2))
    for hbm, vmem in idx_refs:
        pltpu.sync_copy(hbm.at[pl.ds(base, PER_W)], vmem)
    plsc.subcore_barrier()

    lane = lax.iota(jnp.int32, 16)
    perms = [lax.rem(lane + s, 16) for s in (8, 4, 2, 1)]
    gdn = lax.GatherDimensionNumbers(
        offset_dims=(), collapsed_slice_dims=(0,), start_index_map=(0,))

    def lanesum(v):
        for p in perms:
            v = v + lax.gather(v, p[:, None], gdn, slice_sizes=(1,),
                               mode=lax.GatherScatterMode.PROMISE_IN_BOUNDS)
        return v

    def unpack(w):
        lo = plsc.bitcast(lax.shift_left(w, 16), jnp.float32)
        hi = plsc.bitcast(lax.bitwise_and(w, _HI_MASK), jnp.float32)
        return lo, hi

    for r, (isv, idv, oh) in enumerate(((is0, id0, o0), (is1, id1, o1), (is2, id2, o2))):
        relregs = [rel_v[r, pl.ds(16 * j, 16)] for j in range(2 * NT)]

        pltpu.async_copy(shared_tab.at[isv.at[pl.ds(0, C)]], sbufA, semA)
        pltpu.async_copy(shared_tab.at[idv.at[pl.ds(0, C)]], dbufA, semA)

        @pl.loop(0, NCHUNK, step=2)
        def _(c0):
            bufs = ((sbufA, dbufA, semA, sbufB, dbufB, semB),
                    (sbufB, dbufB, semB, sbufA, dbufA, semA))
            for b, (sb, db, sem, osb, odb, osem) in enumerate(bufs):
                c = c0 + b
                pltpu.make_async_copy(shared_tab.at[isv.at[pl.ds(0, C)]], sb, sem).wait()
                pltpu.make_async_copy(shared_tab.at[idv.at[pl.ds(0, C)]], db, sem).wait()

                @pl.when(c + 1 < NCHUNK)
                def _():
                    off = (c + 1) * C
                    pltpu.async_copy(shared_tab.at[isv.at[pl.ds(off, C)]], osb, osem)
                    pltpu.async_copy(shared_tab.at[idv.at[pl.ds(off, C)]], odb, osem)

                def grp(g, carry):
                    acc = jnp.zeros((16,), jnp.float32)
                    for e in range(16):
                        row = g * 16 + e
                        p = None
                        for t in range(NT):
                            slo, shi = unpack(sb[row, pl.ds(16 * t, 16)])
                            dlo, dhi = unpack(db[row, pl.ds(16 * t, 16)])
                            q = slo * dlo * relregs[2 * t] + shi * dhi * relregs[2 * t + 1]
                            p = q if p is None else p + q
                        acc = jnp.where(lane == e, lanesum(p), acc)
                    out_v[pl.ds(c * C + g * 16, 16)] = acc
                    return carry

                lax.fori_loop(0, C // 16, grp, 0)

        pltpu.sync_copy(out_v, oh.at[pl.ds(base, PER_W)])


@jax.jit
def _dis_mult_sc(node_packed, s0, d0, s1, d1, s2, d2, rel_prep):
    f = pl.kernel(
        _body,
        out_type=[jax.ShapeDtypeStruct((E_PAD,), jnp.float32)] * 3,
        mesh=plsc.VectorSubcoreMesh(core_axis_name="c", subcore_axis_name="s",
                                    num_cores=NC, num_subcores=NS),
        scratch_types=[pltpu.VMEM((PER_W,), jnp.int32) for _ in range(6)]
        + [pltpu.VMEM((3, D), jnp.float32), pltpu.VMEM((PER_W,), jnp.float32)]
        + [pltpu.VMEM((C, DW), jnp.int32) for _ in range(4)]
        + [pltpu.VMEM_SHARED((N_NODES, DW), jnp.int32)]
        + [pltpu.SemaphoreType.DMA, pltpu.SemaphoreType.DMA],
        compiler_params=pltpu.CompilerParams(needs_layout_passes=False,
                                             use_tc_tiling_on_sc=False),
    )
    return f(node_packed, s0, d0, s1, d1, s2, d2, rel_prep)


def kernel(node_embeds, edge_index_rel0, edge_index_rel1, edge_index_rel2,
           rel_emb_rel0, rel_emb_rel1, rel_emb_rel2, rel_emb_no_relation):
    pad = E_PAD - E
    flat = []
    for ei in (edge_index_rel0, edge_index_rel1, edge_index_rel2):
        flat.append(jnp.pad(ei[0], (0, pad)))
        flat.append(jnp.pad(ei[1], (0, pad)))
    node_bf = node_embeds.astype(jnp.bfloat16)
    node_packed = lax.bitcast_convert_type(
        node_bf.reshape(N_NODES, DW, 2), jnp.int32)
    rel_all = jnp.concatenate([rel_emb_rel0, rel_emb_rel1, rel_emb_rel2], axis=0)
    rel_prep = rel_all.reshape(3, NT, 16, 2).transpose(0, 1, 3, 2).reshape(3, D)
    outs = _dis_mult_sc(node_packed, *flat, rel_prep)
    return tuple(o[:E] for o in outs)


# speedup vs baseline: 8.6167x; 1.2288x over previous
import numpy as np
import jax
import jax.numpy as jnp
from jax import lax
from jax.experimental import pallas as pl
from jax.experimental.pallas import tpu as pltpu
from jax.experimental.pallas import tpu_sc as plsc

N_NODES = 10000
D = 128
DW = D // 2
E = 100000
NC = 2
NS = 16
NW = NC * NS
PER_W = 3200
E_PAD = NW * PER_W
C = 160
NCHUNK = PER_W // C
NT = DW // 16
OW = 800

_HI_MASK = np.int32(-65536)


def _body(node_h, e0, e1, e2, rel_h,
          o0, o1, o2,
          is0, id0, is1, id1, is2, id2, rel_v, out_v,
          sbufA, dbufA, sbufB, dbufB, shared_tab, semA, semB):
    cid = lax.axis_index("c")
    sid = lax.axis_index("s")
    wid = sid * NC + cid
    base = wid * PER_W

    @pl.when(sid == 0)
    def _():
        pltpu.sync_copy(node_h, shared_tab)

    pltpu.sync_copy(rel_h, rel_v)
    idx_refs = ((e0, is0, id0), (e1, is1, id1), (e2, is2, id2))
    for hbm, vsrc, vdst in idx_refs:
        pltpu.sync_copy(hbm.at[0, pl.ds(base, PER_W)], vsrc)
        pltpu.sync_copy(hbm.at[1, pl.ds(base, PER_W)], vdst)
    plsc.subcore_barrier()

    lane = lax.iota(jnp.int32, 16)
    perms = [lax.rem(lane + s, 16) for s in (8, 4, 2, 1)]
    gdn = lax.GatherDimensionNumbers(
        offset_dims=(), collapsed_slice_dims=(0,), start_index_map=(0,))

    def lanesum(v):
        for p in perms:
            v = v + lax.gather(v, p[:, None], gdn, slice_sizes=(1,),
                               mode=lax.GatherScatterMode.PROMISE_IN_BOUNDS)
        return v

    def unpack(w):
        lo = plsc.bitcast(lax.shift_left(w, 16), jnp.float32)
        hi = plsc.bitcast(lax.bitwise_and(w, _HI_MASK), jnp.float32)
        return lo, hi

    for r, (isv, idv, oh) in enumerate(((is0, id0, o0), (is1, id1, o1), (is2, id2, o2))):
        relregs = [rel_v[r, pl.ds(16 * j, 16)] for j in range(2 * NT)]

        pltpu.async_copy(shared_tab.at[isv.at[pl.ds(0, C)]], sbufA, semA)
        pltpu.async_copy(shared_tab.at[idv.at[pl.ds(0, C)]], dbufA, semA)

        @pl.loop(0, NCHUNK, step=2)
        def _(c0):
            bufs = ((sbufA, dbufA, semA, sbufB, dbufB, semB),
                    (sbufB, dbufB, semB, sbufA, dbufA, semA))
            for b, (sb, db, sem, osb, odb, osem) in enumerate(bufs):
                c = c0 + b
                pltpu.make_async_copy(shared_tab.at[isv.at[pl.ds(0, C)]], sb, sem).wait()
                pltpu.make_async_copy(shared_tab.at[idv.at[pl.ds(0, C)]], db, sem).wait()

                @pl.when(c + 1 < NCHUNK)
                def _():
                    off = (c + 1) * C
                    pltpu.async_copy(shared_tab.at[isv.at[pl.ds(off, C)]], osb, osem)
                    pltpu.async_copy(shared_tab.at[idv.at[pl.ds(off, C)]], odb, osem)

                def grp(g, carry):
                    acc = jnp.zeros((16,), jnp.float32)
                    for e in range(16):
                        row = g * 16 + e
                        p = None
                        for t in range(NT):
                            slo, shi = unpack(sb[row, pl.ds(16 * t, 16)])
                            dlo, dhi = unpack(db[row, pl.ds(16 * t, 16)])
                            q = slo * dlo * relregs[t] + shi * dhi * relregs[t + NT]
                            p = q if p is None else p + q
                        acc = jnp.where(lane == e, lanesum(p), acc)
                    out_v[pl.ds(c * C + g * 16, 16)] = acc
                    return carry

                lax.fori_loop(0, C // 16, grp, 0)

        for k in range(PER_W // OW):
            @pl.when(base + (k + 1) * OW <= E)
            def _():
                pltpu.sync_copy(out_v.at[pl.ds(k * OW, OW)],
                                oh.at[pl.ds(base + k * OW, OW)])


@jax.jit
def _dis_mult_sc(node_packed, e0, e1, e2, rel_all):
    f = pl.kernel(
        _body,
        out_type=[jax.ShapeDtypeStruct((E,), jnp.float32)] * 3,
        mesh=plsc.VectorSubcoreMesh(core_axis_name="c", subcore_axis_name="s",
                                    num_cores=NC, num_subcores=NS),
        scratch_types=[pltpu.VMEM((PER_W,), jnp.int32) for _ in range(6)]
        + [pltpu.VMEM((3, D), jnp.float32), pltpu.VMEM((PER_W,), jnp.float32)]
        + [pltpu.VMEM((C, DW), jnp.int32) for _ in range(4)]
        + [pltpu.VMEM_SHARED((N_NODES, DW), jnp.int32)]
        + [pltpu.SemaphoreType.DMA, pltpu.SemaphoreType.DMA],
        compiler_params=pltpu.CompilerParams(needs_layout_passes=False,
                                             use_tc_tiling_on_sc=False),
    )
    return f(node_packed, e0, e1, e2, rel_all)


def kernel(node_embeds, edge_index_rel0, edge_index_rel1, edge_index_rel2,
           rel_emb_rel0, rel_emb_rel1, rel_emb_rel2, rel_emb_no_relation):
    pad = E_PAD - E
    eis = [jnp.pad(ei, ((0, 0), (0, pad)))
           for ei in (edge_index_rel0, edge_index_rel1, edge_index_rel2)]
    u = lax.bitcast_convert_type(node_embeds, jnp.int32)
    rnd = lax.shift_right_logical(
        u + 0x7FFF + lax.bitwise_and(lax.shift_right_logical(u, 16), 1), 16)
    node_packed = lax.bitwise_or(rnd[:, :DW], lax.shift_left(rnd[:, DW:], 16))
    rel_all = jnp.concatenate([rel_emb_rel0, rel_emb_rel1, rel_emb_rel2], axis=0)
    return tuple(_dis_mult_sc(node_packed, *eis, rel_all))


# speedup vs baseline: 12.3565x; 1.4340x over previous
import numpy as np
import jax
import jax.numpy as jnp
from jax import lax
from jax.experimental import pallas as pl
from jax.experimental.pallas import tpu as pltpu
from jax.experimental.pallas import tpu_sc as plsc

N_NODES = 10000
D = 128
DW = D // 2
E = 100000
NC = 2
NS = 16
NW = NC * NS
PER_W = 3200
E_PAD = NW * PER_W
C = 160
NCHUNK = PER_W // C
NT = DW // 16
OW = 800

_HI_MASK = np.int32(-65536)


def _body(node_h, e0, e1, e2, rel_h,
          o0, o1, o2,
          is0, id0, is1, id1, is2, id2, rel_v, out_v,
          sbufA, dbufA, sbufB, dbufB, shared_tab, semA, semB):
    cid = lax.axis_index("c")
    sid = lax.axis_index("s")
    wid = sid * NC + cid
    base = wid * PER_W

    @pl.when(sid == 0)
    def _():
        pltpu.sync_copy(node_h, shared_tab)

    pltpu.sync_copy(rel_h, rel_v)
    idx_refs = ((e0, is0, id0), (e1, is1, id1), (e2, is2, id2))
    for hbm, vsrc, vdst in idx_refs:
        pltpu.sync_copy(hbm.at[0, pl.ds(base, PER_W)], vsrc)
        pltpu.sync_copy(hbm.at[1, pl.ds(base, PER_W)], vdst)
    plsc.subcore_barrier()

    lane = lax.iota(jnp.int32, 16)
    perms = [lax.rem(lane + s, 16) for s in (8, 4, 2, 1)]
    gdn = lax.GatherDimensionNumbers(
        offset_dims=(), collapsed_slice_dims=(0,), start_index_map=(0,))

    def lanesum(v):
        for p in perms:
            v = v + lax.gather(v, p[:, None], gdn, slice_sizes=(1,),
                               mode=lax.GatherScatterMode.PROMISE_IN_BOUNDS)
        return v

    def pair_prod(sw, dw):
        pr = plsc.bitcast(sw, jnp.bfloat16) * plsc.bitcast(dw, jnp.bfloat16)
        return plsc.unpack(pr, format=plsc.PackFormat.INTERLEAVED,
                           preferred_element_type=jnp.float32)

    for r, (isv, idv, oh) in enumerate(((is0, id0, o0), (is1, id1, o1), (is2, id2, o2))):
        relregs = [rel_v[r, pl.ds(16 * j, 16)] for j in range(2 * NT)]

        pltpu.async_copy(shared_tab.at[isv.at[pl.ds(0, C)]], sbufA, semA)
        pltpu.async_copy(shared_tab.at[idv.at[pl.ds(0, C)]], dbufA, semA)

        @pl.loop(0, NCHUNK, step=2)
        def _(c0):
            bufs = ((sbufA, dbufA, semA, sbufB, dbufB, semB),
                    (sbufB, dbufB, semB, sbufA, dbufA, semA))
            for b, (sb, db, sem, osb, odb, osem) in enumerate(bufs):
                c = c0 + b
                pltpu.make_async_copy(shared_tab.at[isv.at[pl.ds(0, C)]], sb, sem).wait()
                pltpu.make_async_copy(shared_tab.at[idv.at[pl.ds(0, C)]], db, sem).wait()

                @pl.when(c + 1 < NCHUNK)
                def _():
                    off = (c + 1) * C
                    pltpu.async_copy(shared_tab.at[isv.at[pl.ds(off, C)]], osb, osem)
                    pltpu.async_copy(shared_tab.at[idv.at[pl.ds(off, C)]], odb, osem)

                def grp(g, carry):
                    acc = jnp.zeros((16,), jnp.float32)
                    for e in range(16):
                        row = g * 16 + e
                        p = None
                        for t in range(NT):
                            plo, phi = pair_prod(sb[row, pl.ds(16 * t, 16)],
                                                 db[row, pl.ds(16 * t, 16)])
                            q = plo * relregs[t] + phi * relregs[t + NT]
                            p = q if p is None else p + q
                        acc = jnp.where(lane == e, lanesum(p), acc)
                    out_v[pl.ds(c * C + g * 16, 16)] = acc
                    return carry

                lax.fori_loop(0, C // 16, grp, 0)

        for k in range(PER_W // OW):
            @pl.when(base + (k + 1) * OW <= E)
            def _():
                pltpu.sync_copy(out_v.at[pl.ds(k * OW, OW)],
                                oh.at[pl.ds(base + k * OW, OW)])


@jax.jit
def _dis_mult_sc(node_packed, e0, e1, e2, rel_all):
    f = pl.kernel(
        _body,
        out_type=[jax.ShapeDtypeStruct((E,), jnp.float32)] * 3,
        mesh=plsc.VectorSubcoreMesh(core_axis_name="c", subcore_axis_name="s",
                                    num_cores=NC, num_subcores=NS),
        scratch_types=[pltpu.VMEM((PER_W,), jnp.int32) for _ in range(6)]
        + [pltpu.VMEM((3, D), jnp.float32), pltpu.VMEM((PER_W,), jnp.float32)]
        + [pltpu.VMEM((C, DW), jnp.int32) for _ in range(4)]
        + [pltpu.VMEM_SHARED((N_NODES, DW), jnp.int32)]
        + [pltpu.SemaphoreType.DMA, pltpu.SemaphoreType.DMA],
        compiler_params=pltpu.CompilerParams(needs_layout_passes=False,
                                             use_tc_tiling_on_sc=False),
    )
    return f(node_packed, e0, e1, e2, rel_all)


def kernel(node_embeds, edge_index_rel0, edge_index_rel1, edge_index_rel2,
           rel_emb_rel0, rel_emb_rel1, rel_emb_rel2, rel_emb_no_relation):
    pad = E_PAD - E
    eis = [jnp.pad(ei, ((0, 0), (0, pad)))
           for ei in (edge_index_rel0, edge_index_rel1, edge_index_rel2)]
    u = lax.bitcast_convert_type(node_embeds, jnp.int32)
    rnd = lax.shift_right_logical(
        u + 0x7FFF + lax.bitwise_and(lax.shift_right_logical(u, 16), 1), 16)
    node_packed = lax.bitwise_or(rnd[:, :DW], lax.shift_left(rnd[:, DW:], 16))
    rel_all = jnp.concatenate([rel_emb_rel0, rel_emb_rel1, rel_emb_rel2], axis=0)
    return tuple(_dis_mult_sc(node_packed, *eis, rel_all))


# speedup vs baseline: 12.7014x; 1.0279x over previous
import numpy as np
import jax
import jax.numpy as jnp
from jax import lax
from jax.experimental import pallas as pl
from jax.experimental.pallas import tpu as pltpu
from jax.experimental.pallas import tpu_sc as plsc

N_NODES = 10000
D = 128
DW = D // 2
E = 100000
NC = 2
NS = 16
NW = NC * NS
PER_W = 3200
E_PAD = NW * PER_W
C = 160
NCHUNK = PER_W // C
NT = DW // 16
OW = 800

_HI_MASK = np.int32(-65536)


def _body(node_h, e0, e1, e2, rel_h,
          o0, o1, o2,
          is0, id0, is1, id1, is2, id2, rel_v, out_v,
          sbufA, dbufA, sbufB, dbufB, shared_tab, semA, semB):
    cid = lax.axis_index("c")
    sid = lax.axis_index("s")
    wid = sid * NC + cid
    base = wid * PER_W

    @pl.when(sid == 0)
    def _():
        pltpu.sync_copy(node_h, shared_tab)

    pltpu.sync_copy(rel_h, rel_v)
    idx_refs = ((e0, is0, id0), (e1, is1, id1), (e2, is2, id2))
    for hbm, vsrc, vdst in idx_refs:
        pltpu.sync_copy(hbm.at[0, pl.ds(base, PER_W)], vsrc)
        pltpu.sync_copy(hbm.at[1, pl.ds(base, PER_W)], vdst)
    plsc.subcore_barrier()

    lane = lax.iota(jnp.int32, 16)
    perms = [lax.rem(lane + s, 16) for s in (8, 4, 2, 1)]
    gdn = lax.GatherDimensionNumbers(
        offset_dims=(), collapsed_slice_dims=(0,), start_index_map=(0,))

    def lanesum(v):
        for p in perms:
            v = v + lax.gather(v, p[:, None], gdn, slice_sizes=(1,),
                               mode=lax.GatherScatterMode.PROMISE_IN_BOUNDS)
        return v

    def pair_prod(sw, dw, rl):
        pr = plsc.bitcast(sw, jnp.bfloat16) * plsc.bitcast(dw, jnp.bfloat16) * rl
        return plsc.unpack(pr, format=plsc.PackFormat.INTERLEAVED,
                           preferred_element_type=jnp.float32)

    for r, (isv, idv, oh) in enumerate(((is0, id0, o0), (is1, id1, o1), (is2, id2, o2))):
        relregs = [rel_v[r, pl.ds(32 * t, 32)] for t in range(NT)]

        pltpu.async_copy(shared_tab.at[isv.at[pl.ds(0, C)]], sbufA, semA)
        pltpu.async_copy(shared_tab.at[idv.at[pl.ds(0, C)]], dbufA, semA)

        @pl.loop(0, NCHUNK, step=2)
        def _(c0):
            bufs = ((sbufA, dbufA, semA, sbufB, dbufB, semB),
                    (sbufB, dbufB, semB, sbufA, dbufA, semA))
            for b, (sb, db, sem, osb, odb, osem) in enumerate(bufs):
                c = c0 + b
                pltpu.make_async_copy(shared_tab.at[isv.at[pl.ds(0, C)]], sb, sem).wait()
                pltpu.make_async_copy(shared_tab.at[idv.at[pl.ds(0, C)]], db, sem).wait()

                @pl.when(c + 1 < NCHUNK)
                def _():
                    off = (c + 1) * C
                    pltpu.async_copy(shared_tab.at[isv.at[pl.ds(off, C)]], osb, osem)
                    pltpu.async_copy(shared_tab.at[idv.at[pl.ds(off, C)]], odb, osem)

                def grp(g, carry):
                    acc = jnp.zeros((16,), jnp.float32)
                    for e in range(16):
                        row = g * 16 + e
                        p = None
                        for t in range(NT):
                            plo, phi = pair_prod(sb[row, pl.ds(16 * t, 16)],
                                                 db[row, pl.ds(16 * t, 16)],
                                                 relregs[t])
                            q = plo + phi
                            p = q if p is None else p + q
                        acc = jnp.where(lane == e, lanesum(p), acc)
                    out_v[pl.ds(c * C + g * 16, 16)] = acc
                    return carry

                lax.fori_loop(0, C // 16, grp, 0)

        for k in range(PER_W // OW):
            @pl.when(base + (k + 1) * OW <= E)
            def _():
                pltpu.sync_copy(out_v.at[pl.ds(k * OW, OW)],
                                oh.at[pl.ds(base + k * OW, OW)])


@jax.jit
def _dis_mult_sc(node_packed, e0, e1, e2, rel_all):
    f = pl.kernel(
        _body,
        out_type=[jax.ShapeDtypeStruct((E,), jnp.float32)] * 3,
        mesh=plsc.VectorSubcoreMesh(core_axis_name="c", subcore_axis_name="s",
                                    num_cores=NC, num_subcores=NS),
        scratch_types=[pltpu.VMEM((PER_W,), jnp.int32) for _ in range(6)]
        + [pltpu.VMEM((3, D), jnp.bfloat16), pltpu.VMEM((PER_W,), jnp.float32)]
        + [pltpu.VMEM((C, DW), jnp.int32) for _ in range(4)]
        + [pltpu.VMEM_SHARED((N_NODES, DW), jnp.int32)]
        + [pltpu.SemaphoreType.DMA, pltpu.SemaphoreType.DMA],
        compiler_params=pltpu.CompilerParams(needs_layout_passes=False,
                                             use_tc_tiling_on_sc=False),
    )
    return f(node_packed, e0, e1, e2, rel_all)


def kernel(node_embeds, edge_index_rel0, edge_index_rel1, edge_index_rel2,
           rel_emb_rel0, rel_emb_rel1, rel_emb_rel2, rel_emb_no_relation):
    pad = E_PAD - E
    eis = [jnp.pad(ei, ((0, 0), (0, pad)))
           for ei in (edge_index_rel0, edge_index_rel1, edge_index_rel2)]
    u = lax.bitcast_convert_type(node_embeds, jnp.int32)
    rnd = lax.shift_right_logical(
        u + 0x7FFF + lax.bitwise_and(lax.shift_right_logical(u, 16), 1), 16)
    node_packed = lax.bitwise_or(rnd[:, :DW], lax.shift_left(rnd[:, DW:], 16))
    rel_all = jnp.concatenate([rel_emb_rel0, rel_emb_rel1, rel_emb_rel2], axis=0)
    rel_prep = rel_all.reshape(3, 2, NT, 16).transpose(0, 2, 3, 1) \
        .reshape(3, D).astype(jnp.bfloat16)
    return tuple(_dis_mult_sc(node_packed, *eis, rel_prep))


# speedup vs baseline: 12.9786x; 1.0218x over previous
import numpy as np
import jax
import jax.numpy as jnp
from jax import lax
from jax.experimental import pallas as pl
from jax.experimental.pallas import tpu as pltpu
from jax.experimental.pallas import tpu_sc as plsc

N_NODES = 10000
D = 128
DW = D // 2
E = 100000
NC = 2
NS = 16
NW = NC * NS
PER_W = 3200
E_PAD = NW * PER_W
C = 160
NCHUNK = PER_W // C
NT = DW // 16
OW = 800

_HI_MASK = np.int32(-65536)


def _body(node_h, edges_h, rel_h,
          o0, o1, o2,
          is0, id0, is1, id1, is2, id2, rel_v, out_v,
          sbufA, dbufA, sbufB, dbufB, shared_tab, semA, semB):
    cid = lax.axis_index("c")
    sid = lax.axis_index("s")
    wid = sid * NC + cid
    base = wid * PER_W

    rows = N_NODES // NS
    pltpu.sync_copy(node_h.at[pl.ds(sid * rows, rows)],
                    shared_tab.at[pl.ds(sid * rows, rows)])

    pltpu.sync_copy(rel_h, rel_v)
    idx_refs = ((0, is0, id0), (1, is1, id1), (2, is2, id2))
    for r, vsrc, vdst in idx_refs:
        pltpu.sync_copy(edges_h.at[r, 0, pl.ds(base, PER_W)], vsrc)
        pltpu.sync_copy(edges_h.at[r, 1, pl.ds(base, PER_W)], vdst)
    plsc.subcore_barrier()

    lane = lax.iota(jnp.int32, 16)

    def pair_prod(sw, dw, rl):
        pr = plsc.bitcast(sw, jnp.bfloat16) * plsc.bitcast(dw, jnp.bfloat16) * rl
        return plsc.unpack(pr, format=plsc.PackFormat.INTERLEAVED,
                           preferred_element_type=jnp.float32)

    for r, (isv, idv, oh) in enumerate(((is0, id0, o0), (is1, id1, o1), (is2, id2, o2))):
        relregs = [rel_v[r, pl.ds(32 * t, 32)] for t in range(NT)]

        pltpu.async_copy(shared_tab.at[isv.at[pl.ds(0, C)]], sbufA, semA)
        pltpu.async_copy(shared_tab.at[idv.at[pl.ds(0, C)]], dbufA, semA)

        @pl.loop(0, NCHUNK, step=2)
        def _(c0):
            bufs = ((sbufA, dbufA, semA, sbufB, dbufB, semB),
                    (sbufB, dbufB, semB, sbufA, dbufA, semA))
            for b, (sb, db, sem, osb, odb, osem) in enumerate(bufs):
                c = c0 + b
                pltpu.make_async_copy(shared_tab.at[isv.at[pl.ds(0, C)]], sb, sem).wait()
                pltpu.make_async_copy(shared_tab.at[idv.at[pl.ds(0, C)]], db, sem).wait()

                @pl.when(c + 1 < NCHUNK)
                def _():
                    off = (c + 1) * C
                    pltpu.async_copy(shared_tab.at[isv.at[pl.ds(off, C)]], osb, osem)
                    pltpu.async_copy(shared_tab.at[idv.at[pl.ds(off, C)]], odb, osem)

                def grp(g, carry):
                    acc = jnp.zeros((16,), jnp.float32)
                    for e in range(16):
                        row = g * 16 + e
                        p = None
                        for t in range(NT):
                            plo, phi = pair_prod(sb[row, pl.ds(16 * t, 16)],
                                                 db[row, pl.ds(16 * t, 16)],
                                                 relregs[t])
                            q = plo + phi
                            p = q if p is None else p + q
                        acc = jnp.where(lane == e, jnp.sum(p), acc)
                    out_v[pl.ds(c * C + g * 16, 16)] = acc
                    return carry

                lax.fori_loop(0, C // 16, grp, 0)

        for k in range(PER_W // OW):
            @pl.when(base + (k + 1) * OW <= E)
            def _():
                pltpu.sync_copy(out_v.at[pl.ds(k * OW, OW)],
                                oh.at[pl.ds(base + k * OW, OW)])


@jax.jit
def _dis_mult_sc(node_packed, edges, rel_all):
    f = pl.kernel(
        _body,
        out_type=[jax.ShapeDtypeStruct((E,), jnp.float32)] * 3,
        mesh=plsc.VectorSubcoreMesh(core_axis_name="c", subcore_axis_name="s",
                                    num_cores=NC, num_subcores=NS),
        scratch_types=[pltpu.VMEM((PER_W,), jnp.int32) for _ in range(6)]
        + [pltpu.VMEM((3, D), jnp.bfloat16), pltpu.VMEM((PER_W,), jnp.float32)]
        + [pltpu.VMEM((C, DW), jnp.int32) for _ in range(4)]
        + [pltpu.VMEM_SHARED((N_NODES, DW), jnp.int32)]
        + [pltpu.SemaphoreType.DMA, pltpu.SemaphoreType.DMA],
        compiler_params=pltpu.CompilerParams(needs_layout_passes=False,
                                             use_tc_tiling_on_sc=False),
    )
    return f(node_packed, edges, rel_all)


def kernel(node_embeds, edge_index_rel0, edge_index_rel1, edge_index_rel2,
           rel_emb_rel0, rel_emb_rel1, rel_emb_rel2, rel_emb_no_relation):
    pad = E_PAD - E
    edges = jnp.pad(
        jnp.stack((edge_index_rel0, edge_index_rel1, edge_index_rel2)),
        ((0, 0), (0, 0), (0, pad)))
    u = lax.bitcast_convert_type(node_embeds, jnp.int32)
    rnd = lax.shift_right_logical(
        u + 0x7FFF + lax.bitwise_and(lax.shift_right_logical(u, 16), 1), 16)
    node_packed = lax.bitwise_or(rnd[:, :DW], lax.shift_left(rnd[:, DW:], 16))
    rel_all = jnp.concatenate([rel_emb_rel0, rel_emb_rel1, rel_emb_rel2], axis=0)
    rel_prep = rel_all.reshape(3, 2, NT, 16).transpose(0, 2, 3, 1) \
        .reshape(3, D).astype(jnp.bfloat16)
    return tuple(_dis_mult_sc(node_packed, edges, rel_prep))


# speedup vs baseline: 13.4500x; 1.0363x over previous
import numpy as np
import jax
import jax.numpy as jnp
from jax import lax
from jax.experimental import pallas as pl
from jax.experimental.pallas import tpu as pltpu
from jax.experimental.pallas import tpu_sc as plsc

N_NODES = 10000
D = 128
DW = D // 2
E = 100000
NC = 2
NS = 16
NW = NC * NS
PER_W = 3200
E_PAD = NW * PER_W
C = 160
NCHUNK = PER_W // C
NT = DW // 16
OW = 800

_HI_MASK = np.int32(-65536)


def _body(node_h, edges_h, rel_h,
          o0, o1, o2,
          is0, id0, is1, id1, is2, id2, rel_v, out_v,
          sbufA, dbufA, sbufB, dbufB, shared_tab, semA, semB):
    cid = lax.axis_index("c")
    sid = lax.axis_index("s")
    wid = sid * NC + cid
    base = wid * PER_W

    rows = N_NODES // NS
    entry = [pltpu.async_copy(node_h.at[pl.ds(sid * rows, rows)],
                              shared_tab.at[pl.ds(sid * rows, rows)], semB),
             pltpu.async_copy(rel_h, rel_v, semB)]
    for r, vsrc, vdst in ((0, is0, id0), (1, is1, id1), (2, is2, id2)):
        entry.append(pltpu.async_copy(edges_h.at[r, 0, pl.ds(base, PER_W)], vsrc, semB))
        entry.append(pltpu.async_copy(edges_h.at[r, 1, pl.ds(base, PER_W)], vdst, semB))
    for h in entry:
        h.wait()
    plsc.subcore_barrier()

    lane = lax.iota(jnp.int32, 16)

    def pair_prod(sw, dw, rl):
        pr = plsc.bitcast(sw, jnp.bfloat16) * plsc.bitcast(dw, jnp.bfloat16) * rl
        return plsc.unpack(pr, format=plsc.PackFormat.INTERLEAVED,
                           preferred_element_type=jnp.float32)

    rels = ((is0, id0, o0), (is1, id1, o1), (is2, id2, o2))
    pltpu.async_copy(shared_tab.at[is0.at[pl.ds(0, C)]], sbufA, semA)
    pltpu.async_copy(shared_tab.at[id0.at[pl.ds(0, C)]], dbufA, semA)
    for r, (isv, idv, oh) in enumerate(rels):
        relregs = [rel_v[r, pl.ds(32 * t, 32)] for t in range(NT)]

        @pl.loop(0, NCHUNK, step=2)
        def _(c0):
            bufs = ((sbufA, dbufA, semA, sbufB, dbufB, semB),
                    (sbufB, dbufB, semB, sbufA, dbufA, semA))
            for b, (sb, db, sem, osb, odb, osem) in enumerate(bufs):
                c = c0 + b
                pltpu.make_async_copy(shared_tab.at[isv.at[pl.ds(0, C)]], sb, sem).wait()
                pltpu.make_async_copy(shared_tab.at[idv.at[pl.ds(0, C)]], db, sem).wait()

                @pl.when(c + 1 < NCHUNK)
                def _():
                    off = (c + 1) * C
                    pltpu.async_copy(shared_tab.at[isv.at[pl.ds(off, C)]], osb, osem)
                    pltpu.async_copy(shared_tab.at[idv.at[pl.ds(off, C)]], odb, osem)

                def grp(g, carry):
                    acc = jnp.zeros((16,), jnp.float32)
                    for e in range(16):
                        row = g * 16 + e
                        p = None
                        for t in range(NT):
                            plo, phi = pair_prod(sb[row, pl.ds(16 * t, 16)],
                                                 db[row, pl.ds(16 * t, 16)],
                                                 relregs[t])
                            q = plo + phi
                            p = q if p is None else p + q
                        acc = jnp.where(lane == e, jnp.sum(p), acc)
                    out_v[pl.ds(c * C + g * 16, 16)] = acc
                    return carry

                lax.fori_loop(0, C // 16, grp, 0)

        if r + 1 < 3:
            nsv, ndv, _ = rels[r + 1]
            pltpu.async_copy(shared_tab.at[nsv.at[pl.ds(0, C)]], sbufA, semA)
            pltpu.async_copy(shared_tab.at[ndv.at[pl.ds(0, C)]], dbufA, semA)

        for k in range(PER_W // OW):
            @pl.when(base + (k + 1) * OW <= E)
            def _():
                pltpu.sync_copy(out_v.at[pl.ds(k * OW, OW)],
                                oh.at[pl.ds(base + k * OW, OW)])


@jax.jit
def _dis_mult_sc(node_packed, edges, rel_all):
    f = pl.kernel(
        _body,
        out_type=[jax.ShapeDtypeStruct((E,), jnp.float32)] * 3,
        mesh=plsc.VectorSubcoreMesh(core_axis_name="c", subcore_axis_name="s",
                                    num_cores=NC, num_subcores=NS),
        scratch_types=[pltpu.VMEM((PER_W,), jnp.int32) for _ in range(6)]
        + [pltpu.VMEM((3, D), jnp.bfloat16), pltpu.VMEM((PER_W,), jnp.float32)]
        + [pltpu.VMEM((C, DW), jnp.int32) for _ in range(4)]
        + [pltpu.VMEM_SHARED((N_NODES, DW), jnp.int32)]
        + [pltpu.SemaphoreType.DMA, pltpu.SemaphoreType.DMA],
        compiler_params=pltpu.CompilerParams(needs_layout_passes=False,
                                             use_tc_tiling_on_sc=False),
    )
    return f(node_packed, edges, rel_all)


def kernel(node_embeds, edge_index_rel0, edge_index_rel1, edge_index_rel2,
           rel_emb_rel0, rel_emb_rel1, rel_emb_rel2, rel_emb_no_relation):
    pad = E_PAD - E
    edges = jnp.pad(
        jnp.stack((edge_index_rel0, edge_index_rel1, edge_index_rel2)),
        ((0, 0), (0, 0), (0, pad)))
    def _rne16(u):
        return lax.shift_right_logical(
            u + 0x7FFF + lax.bitwise_and(lax.shift_right_logical(u, 16), 1), 16)

    u = lax.bitcast_convert_type(node_embeds, jnp.int32)
    node_packed = lax.bitwise_or(_rne16(u[:, :DW]),
                                 lax.shift_left(_rne16(u[:, DW:]), 16))
    rel_all = jnp.concatenate([rel_emb_rel0, rel_emb_rel1, rel_emb_rel2], axis=0)
    rel_prep = rel_all.reshape(3, 2, NT, 16).transpose(0, 2, 3, 1) \
        .reshape(3, D).astype(jnp.bfloat16)
    return tuple(_dis_mult_sc(node_packed, edges, rel_prep))


# speedup vs baseline: 13.5622x; 1.0083x over previous
import jax
import jax.numpy as jnp
from jax import lax
from jax.experimental import pallas as pl
from jax.experimental.pallas import tpu as pltpu
from jax.experimental.pallas import tpu_sc as plsc

N_NODES = 10000
D = 128
DW = D // 2
E = 100000
NC = 2
NS = 16
NW = NC * NS
PER_W = 3200
E_PAD = NW * PER_W
C = 160
NCHUNK = PER_W // C
NT = DW // 16
OW = 800


def _body(node_h, edges_h, rel_h,
          o0, o1, o2,
          is0, id0, is1, id1, is2, id2, rel_v, out_v,
          sbufA, dbufA, sbufB, dbufB, shared_tab, semA, semB):
    cid = lax.axis_index("c")
    sid = lax.axis_index("s")
    wid = sid * NC + cid
    base = wid * PER_W

    rows = N_NODES // NS
    entry = [pltpu.async_copy(node_h.at[pl.ds(sid * rows, rows)],
                              shared_tab.at[pl.ds(sid * rows, rows)], semB),
             pltpu.async_copy(rel_h, rel_v, semB)]
    for r, vsrc, vdst in ((0, is0, id0), (1, is1, id1), (2, is2, id2)):
        entry.append(pltpu.async_copy(edges_h.at[r, 0, pl.ds(base, PER_W)], vsrc, semB))
        entry.append(pltpu.async_copy(edges_h.at[r, 1, pl.ds(base, PER_W)], vdst, semB))
    for h in entry:
        h.wait()
    plsc.subcore_barrier()

    lane = lax.iota(jnp.int32, 16)

    def pair_prod(sw, dw, rl):
        pr = plsc.bitcast(sw, jnp.bfloat16) * plsc.bitcast(dw, jnp.bfloat16) * rl
        return plsc.unpack(pr, format=plsc.PackFormat.INTERLEAVED,
                           preferred_element_type=jnp.float32)

    rels = ((is0, id0, o0), (is1, id1, o1), (is2, id2, o2))
    pltpu.async_copy(shared_tab.at[is0.at[pl.ds(0, C)]], sbufA, semA)
    pltpu.async_copy(shared_tab.at[id0.at[pl.ds(0, C)]], dbufA, semA)
    for r, (isv, idv, oh) in enumerate(rels):
        relregs = [rel_v[r, pl.ds(32 * t, 32)] for t in range(NT)]

        @pl.loop(0, NCHUNK, step=2)
        def _(c0):
            bufs = ((sbufA, dbufA, semA, sbufB, dbufB, semB),
                    (sbufB, dbufB, semB, sbufA, dbufA, semA))
            for b, (sb, db, sem, osb, odb, osem) in enumerate(bufs):
                c = c0 + b
                pltpu.make_async_copy(shared_tab.at[isv.at[pl.ds(0, C)]], sb, sem).wait()
                pltpu.make_async_copy(shared_tab.at[idv.at[pl.ds(0, C)]], db, sem).wait()

                @pl.when(c + 1 < NCHUNK)
                def _():
                    off = (c + 1) * C
                    pltpu.async_copy(shared_tab.at[isv.at[pl.ds(off, C)]], osb, osem)
                    pltpu.async_copy(shared_tab.at[idv.at[pl.ds(off, C)]], odb, osem)

                def grp(g, carry):
                    acc = jnp.zeros((16,), jnp.float32)
                    for e in range(16):
                        row = g * 16 + e
                        p = None
                        for t in range(NT):
                            plo, phi = pair_prod(sb[row, pl.ds(16 * t, 16)],
                                                 db[row, pl.ds(16 * t, 16)],
                                                 relregs[t])
                            q = plo + phi
                            p = q if p is None else p + q
                        acc = jnp.where(lane == e, jnp.sum(p), acc)
                    out_v[pl.ds(c * C + g * 16, 16)] = acc
                    return carry

                lax.fori_loop(0, C // 16, grp, 0)

        if r + 1 < 3:
            nsv, ndv, _ = rels[r + 1]
            pltpu.async_copy(shared_tab.at[nsv.at[pl.ds(0, C)]], sbufA, semA)
            pltpu.async_copy(shared_tab.at[ndv.at[pl.ds(0, C)]], dbufA, semA)

        for k in range(PER_W // OW):
            @pl.when(base + (k + 1) * OW <= E)
            def _():
                pltpu.sync_copy(out_v.at[pl.ds(k * OW, OW)],
                                oh.at[pl.ds(base + k * OW, OW)])


@jax.jit
def _dis_mult_sc(node_packed, edges, rel_all):
    f = pl.kernel(
        _body,
        out_type=[jax.ShapeDtypeStruct((E,), jnp.float32)] * 3,
        mesh=plsc.VectorSubcoreMesh(core_axis_name="c", subcore_axis_name="s",
                                    num_cores=NC, num_subcores=NS),
        scratch_types=[pltpu.VMEM((PER_W,), jnp.int32) for _ in range(6)]
        + [pltpu.VMEM((3, D), jnp.bfloat16), pltpu.VMEM((PER_W,), jnp.float32)]
        + [pltpu.VMEM((C, DW), jnp.int32) for _ in range(4)]
        + [pltpu.VMEM_SHARED((N_NODES, DW), jnp.int32)]
        + [pltpu.SemaphoreType.DMA, pltpu.SemaphoreType.DMA],
        compiler_params=pltpu.CompilerParams(needs_layout_passes=False,
                                             use_tc_tiling_on_sc=False),
    )
    return f(node_packed, edges, rel_all)


def kernel(node_embeds, edge_index_rel0, edge_index_rel1, edge_index_rel2,
           rel_emb_rel0, rel_emb_rel1, rel_emb_rel2, rel_emb_no_relation):
    pad = E_PAD - E
    edges = jnp.pad(
        jnp.stack((edge_index_rel0, edge_index_rel1, edge_index_rel2)),
        ((0, 0), (0, 0), (0, pad)))
    def _rne16(u):
        return lax.shift_right_logical(
            u + 0x7FFF + lax.bitwise_and(lax.shift_right_logical(u, 16), 1), 16)

    u = lax.bitcast_convert_type(node_embeds, jnp.int32)
    node_packed = lax.bitwise_or(_rne16(u[:, :DW]),
                                 lax.shift_left(_rne16(u[:, DW:]), 16))
    rel_all = jnp.concatenate([rel_emb_rel0, rel_emb_rel1, rel_emb_rel2], axis=0)
    rel_prep = rel_all.reshape(3, 2, NT, 16).transpose(0, 2, 3, 1) \
        .reshape(3, D).astype(jnp.bfloat16)
    return tuple(_dis_mult_sc(node_packed, edges, rel_prep))
